# baseline (speedup 1.0000x reference)
import functools

import jax
import jax.numpy as jnp
from jax import lax
from jax.experimental import pallas as pl
from jax.experimental.pallas import tpu as pltpu
from jax.experimental.pallas import tpu_sc as plsc

_B = 16384
_D = 64
_L = 16
_NW = 32
_BPW = _B // _NW
_G = _BPW // _L
_NC = 2
_CW = _BPW // _NC
_GPC = _CW // _L
_Nmain = 999936
_BC = 128
_NBC = _BPW // _BC

_mesh = plsc.VectorSubcoreMesh(core_axis_name="c", subcore_axis_name="s")


@functools.partial(
    pl.kernel,
    mesh=_mesh,
    out_type=(
        jax.ShapeDtypeStruct((_B,), jnp.float32),
        jax.ShapeDtypeStruct((_NW, _L), jnp.float32),
    ),
    scratch_types=[
        pltpu.VMEM((_BPW,), jnp.int32),
        pltpu.VMEM((_BPW,), jnp.int32),
        [pltpu.VMEM((_CW,), jnp.int32) for _ in range(_NC)],
        [pltpu.VMEM((_CW,), jnp.int32) for _ in range(_NC)],
        [pltpu.VMEM((_BC,), jnp.int32) for _ in range(_NBC)],
        [pltpu.VMEM((_BC,), jnp.int32) for _ in range(_NBC)],
        pltpu.VMEM((_BPW,), jnp.float32),
        pltpu.VMEM((_CW, 128), jnp.float32),
        pltpu.VMEM((_CW, 128), jnp.float32),
        pltpu.VMEM((_BC, 128), jnp.float32),
        pltpu.VMEM((_D,), jnp.float32),
        pltpu.VMEM((_D,), jnp.float32),
        pltpu.VMEM((_BPW,), jnp.float32),
        pltpu.VMEM((_BPW,), jnp.float32),
        pltpu.VMEM((_BPW,), jnp.float32),
        pltpu.VMEM((_L,), jnp.float32),
        pltpu.VMEM((_L,), jnp.float32),
        pltpu.SemaphoreType.DMA,
        pltpu.SemaphoreType.DMA,
    ],
    compiler_params=pltpu.CompilerParams(needs_layout_passes=False),
)
def _mf_sc(du_hbm, di_hbm, dr_hbm, uw2_hbm, iw2_hbm,
           ubR_hbm, ubT_hbm, ibR_hbm, ibT_hbm, bias_hbm,
           pred_hbm, part_hbm,
           uidx_v, iidx_v, urow_v, irow_v, ubrow_v, ibrow_v,
           rat_v, ust_v, ist_v, bst_v, ubt_v, ibt_v, ub_v, ib_v,
           pred_v, acc_v, bias_v, sem, bsem):
    wid = lax.axis_index("s") * 2 + lax.axis_index("c")
    base = wid * _BPW

    pltpu.sync_copy(du_hbm.at[pl.ds(base, _BPW)], uidx_v)
    pltpu.sync_copy(di_hbm.at[pl.ds(base, _BPW)], iidx_v)
    pltpu.sync_copy(dr_hbm.at[pl.ds(base, _BPW)], rat_v)
    pltpu.sync_copy(bias_hbm, bias_v)
    pltpu.sync_copy(ubT_hbm, ubt_v)
    pltpu.sync_copy(ibT_hbm, ibt_v)

    iota = lax.iota(jnp.int32, _L)

    for k in range(_G):
        sl = pl.ds(k * _L, _L)
        un = uidx_v[sl]
        it = iidx_v[sl]
        n = k * _L
        urow_v[n // _CW][pl.ds(n % _CW, _L)] = lax.shift_right_logical(un, 1)
        irow_v[n // _CW][pl.ds(n % _CW, _L)] = lax.shift_right_logical(it, 1)
        ubrow_v[n // _BC][pl.ds(n % _BC, _L)] = jnp.minimum(
            lax.shift_right_logical(un, 7), 7811)
        ibrow_v[n // _BC][pl.ds(n % _BC, _L)] = jnp.minimum(
            lax.shift_right_logical(it, 7), 7811)

    for tbl in range(2):
        rows_v = ubrow_v if tbl == 0 else ibrow_v
        src = ubR_hbm if tbl == 0 else ibR_hbm
        idxfull = uidx_v if tbl == 0 else iidx_v
        tail_v = ubt_v if tbl == 0 else ibt_v
        out_v = ub_v if tbl == 0 else ib_v
        for c in range(_NBC):
            pltpu.async_copy(src.at[rows_v[c]], bst_v, bsem).wait()
            for k in range(_BC // _L):
                n0 = c * _BC + k * _L
                iv = idxfull[pl.ds(n0, _L)]
                col = lax.bitwise_and(iv, 127)
                row = k * _L + iota
                val = plsc.load_gather(bst_v, [row, col])
                toff = jnp.clip(iv - _Nmain, 0, _D - 1)
                tval = plsc.load_gather(tail_v, [toff])
                val = jnp.where(iv >= _Nmain, tval, val)
                out_v[pl.ds(n0, _L)] = val

    bias_vec = bias_v[...]

    acc = jnp.zeros((_L,), jnp.float32)
    for c in range(_NC):
        cu = pltpu.async_copy(uw2_hbm.at[urow_v[c]], ust_v, sem)
        ci = pltpu.async_copy(iw2_hbm.at[irow_v[c]], ist_v, sem)
        cu.wait()
        ci.wait()

        def group(gl, acc, c=c):
            n0 = c * _CW
            sl = pl.ds(n0 + gl * _L, _L)
            iu = uidx_v[sl]
            ii = iidx_v[sl]
            rloc = gl * _L + iota
            ou = lax.bitwise_and(iu, 1) * _D
            oi = lax.bitwise_and(ii, 1) * _D
            s = su = si = None
            for j in range(_D):
                u = plsc.load_gather(ust_v, [rloc, ou + j])
                i = plsc.load_gather(ist_v, [rloc, oi + j])
                if s is None:
                    s, su, si = u * i, u, i
                else:
                    s, su, si = s + u * i, su + u, si + i
            ub_g = ub_v[sl]
            ib_g = ib_v[sl]
            pred_g = (s + ub_g * si + ib_g * su
                      + (ub_g * ib_g) * float(_D) + bias_vec)
            pred_v[sl] = pred_g
            err = pred_g - rat_v[sl]
            return acc + err * err

        acc = lax.fori_loop(0, _GPC, group, acc)

    acc_v[...] = acc
    pltpu.sync_copy(pred_v, pred_hbm.at[pl.ds(base, _BPW)])
    pltpu.sync_copy(acc_v, part_hbm.at[wid])


def _loss_body(part_ref, o_ref):
    o_ref[...] = jnp.sum(part_ref[...]).reshape(1, 1) * (1.0 / _B)


@jax.jit
def kernel(data_user, data_item, data_rating, user_weight, item_weight,
           user_bias, item_bias, bias):
    uw2 = user_weight.reshape(500000, 128)
    iw2 = item_weight.reshape(500000, 128)
    ubf = user_bias.reshape(-1)
    ibf = item_bias.reshape(-1)
    ubR = ubf[:_Nmain].reshape(_Nmain // 128, 128)
    ibR = ibf[:_Nmain].reshape(_Nmain // 128, 128)
    ubT = ubf[_Nmain:]
    ibT = ibf[_Nmain:]
    bias16 = jnp.broadcast_to(bias, (_L,))
    pred, partials = _mf_sc(data_user, data_item, data_rating, uw2, iw2,
                            ubR, ubT, ibR, ibT, bias16)
    loss2 = pl.pallas_call(
        _loss_body,
        out_shape=jax.ShapeDtypeStruct((1, 1), jnp.float32),
    )(partials)
    return pred, loss2[0, 0]
